# Initial kernel scaffold; baseline (speedup 1.0000x reference)
import jax
import jax.numpy as jnp
from jax.experimental import pallas as pl


def kernel(x, w_0, w2_0, w_1, w2_1):
    raise NotImplementedError("write your pallas kernel here")


# speedup vs baseline: 5.5254x; 5.5254x over previous
import functools

import jax
import jax.numpy as jnp
from jax import lax
from jax.experimental import pallas as pl
from jax.experimental.pallas import tpu as pltpu
from jax.experimental.pallas import tpu_sc as plsc

_N = 4096
_K = 16
_BLK = 256


def _knn_body(x_ref, xt_ref, out_ref):
    xb = x_ref[...]
    xt = xt_ref[...]
    g = jnp.dot(xb, xt, preferred_element_type=jnp.float32)
    sqi = jnp.sum(xb * xb, axis=1, keepdims=True)
    sqj = jnp.sum(xt * xt, axis=0, keepdims=True)
    d2 = jnp.maximum(sqi + sqj - 2.0 * g, 0.0)
    colid = lax.broadcasted_iota(jnp.int32, d2.shape, 1)
    cols = []
    for _ in range(_K):
        m = jnp.min(d2, axis=1, keepdims=True)
        idx = jnp.min(jnp.where(d2 == m, colid, _N), axis=1, keepdims=True)
        cols.append(idx)
        d2 = jnp.where(colid == idx, jnp.float32(jnp.inf), d2)
    out_ref[...] = jnp.concatenate(cols, axis=1)


def _knn(x, xt):
    n, d = x.shape
    return pl.pallas_call(
        _knn_body,
        grid=(n // _BLK,),
        in_specs=[
            pl.BlockSpec((_BLK, d), lambda i: (i, 0)),
            pl.BlockSpec((d, n), lambda i: (0, 0)),
        ],
        out_specs=pl.BlockSpec((_BLK, _K), lambda i: (i, 0)),
        out_shape=jax.ShapeDtypeStruct((n, _K), jnp.int32),
    )(x, xt)


def _mm_body(a_ref, b_ref, out_ref):
    out_ref[...] = jnp.dot(a_ref[...], b_ref[...],
                           preferred_element_type=jnp.float32)


def _mm(a, b):
    n, ka = a.shape
    kb = b.shape[1]
    return pl.pallas_call(
        _mm_body,
        grid=(n // _BLK,),
        in_specs=[
            pl.BlockSpec((_BLK, ka), lambda i: (i, 0)),
            pl.BlockSpec((ka, kb), lambda i: (0, 0)),
        ],
        out_specs=pl.BlockSpec((_BLK, kb), lambda i: (i, 0)),
        out_shape=jax.ShapeDtypeStruct((n, kb), jnp.float32),
    )(a, b)


def _stage2_body(p_ref, add_ref, m1_ref, m2_ref, out_ref):
    h = jnp.dot(p_ref[...], m1_ref[...],
                preferred_element_type=jnp.float32) + add_ref[...]
    out_ref[...] = jnp.dot(h, m2_ref[...], preferred_element_type=jnp.float32)


def _stage2(pooled, add, m1, m2):
    n, hd = pooled.shape
    d = m1.shape[1]
    kb = m2.shape[1]
    return pl.pallas_call(
        _stage2_body,
        grid=(n // _BLK,),
        in_specs=[
            pl.BlockSpec((_BLK, hd), lambda i: (i, 0)),
            pl.BlockSpec((_BLK, d), lambda i: (i, 0)),
            pl.BlockSpec((hd, d), lambda i: (0, 0)),
            pl.BlockSpec((d, kb), lambda i: (0, 0)),
        ],
        out_specs=pl.BlockSpec((_BLK, kb), lambda i: (i, 0)),
        out_shape=jax.ShapeDtypeStruct((n, kb), jnp.float32),
    )(pooled, add, m1, m2)


def _final_body(p_ref, add_ref, m1_ref, out_ref):
    out_ref[...] = jnp.dot(p_ref[...], m1_ref[...],
                           preferred_element_type=jnp.float32) + add_ref[...]


def _final(pooled, add, m1):
    n, hd = pooled.shape
    d = m1.shape[1]
    return pl.pallas_call(
        _final_body,
        grid=(n // _BLK,),
        in_specs=[
            pl.BlockSpec((_BLK, hd), lambda i: (i, 0)),
            pl.BlockSpec((_BLK, d), lambda i: (i, 0)),
            pl.BlockSpec((hd, d), lambda i: (0, 0)),
        ],
        out_specs=pl.BlockSpec((_BLK, d), lambda i: (i, 0)),
        out_shape=jax.ShapeDtypeStruct((n, d), jnp.float32),
    )(pooled, add, m1)


def _sc_pool(yn, ys, gflat):
    n, h = ys.shape
    info = plsc.get_sparse_core_info()
    nc, ns = info.num_cores, info.num_subcores
    nw = nc * ns
    nodes_per_w = n // nw
    c_nodes = 8
    n_chunks = nodes_per_w // c_nodes
    mesh = plsc.VectorSubcoreMesh(core_axis_name="c", subcore_axis_name="s")

    @functools.partial(
        pl.kernel,
        out_type=jax.ShapeDtypeStruct((n, h), jnp.float32),
        mesh=mesh,
        scratch_types=[
            pltpu.VMEM((c_nodes * _K,), jnp.int32),
            pltpu.VMEM((c_nodes * _K, h), jnp.float32),
            pltpu.VMEM((c_nodes, h), jnp.float32),
            pltpu.VMEM((c_nodes, h), jnp.float32),
            pltpu.SemaphoreType.DMA,
        ],
    )
    def k(yn_hbm, ys_hbm, g_hbm, out_hbm, idx_v, rows_v, ys_v, out_v, sem):
        wid = lax.axis_index("s") * nc + lax.axis_index("c")
        base_node = wid * nodes_per_w

        def chunk_body(ci, carry):
            node0 = base_node + ci * c_nodes
            pltpu.sync_copy(g_hbm.at[pl.ds(node0 * _K, c_nodes * _K)], idx_v)
            pltpu.async_copy(yn_hbm.at[idx_v], rows_v, sem).wait()
            pltpu.sync_copy(ys_hbm.at[pl.ds(node0, c_nodes)], ys_v)

            def node_body(ni, inner):
                for c in range(h // 16):
                    sl = pl.ds(c * 16, 16)
                    yv = ys_v[ni, sl]
                    acc = jnp.zeros((16,), jnp.float32)
                    for j in range(_K):
                        v = rows_v[ni * _K + j, sl]
                        acc = acc + jnp.clip(v + yv, -1.0, 1.0)
                    out_v[ni, sl] = acc * (1.0 / _K)
                return inner

            lax.fori_loop(0, c_nodes, node_body, 0)
            pltpu.sync_copy(out_v, out_hbm.at[pl.ds(node0, c_nodes)])
            return carry

        lax.fori_loop(0, n_chunks, chunk_body, 0)

    return k(yn, ys, gflat)


def kernel(x, w_0, w2_0, w_1, w2_1):
    n, d = x.shape
    hid = w_0.shape[0]

    w0cat = jnp.concatenate(
        [w_0[:, :d].T, w_0[:, d:].T, w2_0[:, hid:].T], axis=1)
    out0 = w2_0.shape[0]
    w1cat = jnp.concatenate(
        [w_1[:, :out0].T, w_1[:, out0:].T, w2_1[:, hid:].T], axis=1)
    w2p0t = w2_0[:, :hid].T
    w2p1t = w2_1[:, :hid].T

    graph = _knn(x, x.T)
    gflat = graph.reshape(-1)

    y0 = _mm(x, w0cat)
    pooled0 = _sc_pool(y0[:, :hid], y0[:, hid:2 * hid], gflat)
    y1 = _stage2(pooled0, y0[:, 2 * hid:], w2p0t, w1cat)
    pooled1 = _sc_pool(y1[:, :hid], y1[:, hid:2 * hid], gflat)
    return _final(pooled1, y1[:, 2 * hid:], w2p1t)


# speedup vs baseline: 5.9959x; 1.0851x over previous
import functools

import jax
import jax.numpy as jnp
from jax import lax
from jax.experimental import pallas as pl
from jax.experimental.pallas import tpu as pltpu
from jax.experimental.pallas import tpu_sc as plsc

_N = 4096
_K = 16
_BLK = 256


def _knn_body(x_ref, xt_ref, out_ref):
    xb = x_ref[...]
    xt = xt_ref[...]
    g = jnp.dot(xb, xt, preferred_element_type=jnp.float32)
    sqi = jnp.sum(xb * xb, axis=1, keepdims=True)
    sqj = jnp.sum(xt * xt, axis=0, keepdims=True)
    d2 = jnp.maximum(sqi + sqj - 2.0 * g, 0.0)
    colid = lax.broadcasted_iota(jnp.int32, d2.shape, 1)
    cols = []
    for _ in range(_K):
        m = jnp.min(d2, axis=1, keepdims=True)
        idx = jnp.min(jnp.where(d2 == m, colid, _N), axis=1, keepdims=True)
        cols.append(idx)
        d2 = jnp.where(colid == idx, jnp.float32(jnp.inf), d2)
    out_ref[...] = jnp.concatenate(cols, axis=1)


def _knn(x, xt):
    n, d = x.shape
    return pl.pallas_call(
        _knn_body,
        grid=(n // _BLK,),
        in_specs=[
            pl.BlockSpec((_BLK, d), lambda i: (i, 0)),
            pl.BlockSpec((d, n), lambda i: (0, 0)),
        ],
        out_specs=pl.BlockSpec((_BLK, _K), lambda i: (i, 0)),
        out_shape=jax.ShapeDtypeStruct((n, _K), jnp.int32),
    )(x, xt)


def _mm_body(a_ref, b_ref, out_ref):
    out_ref[...] = jnp.dot(a_ref[...], b_ref[...],
                           preferred_element_type=jnp.float32)


def _mm(a, b):
    n, ka = a.shape
    kb = b.shape[1]
    return pl.pallas_call(
        _mm_body,
        grid=(n // _BLK,),
        in_specs=[
            pl.BlockSpec((_BLK, ka), lambda i: (i, 0)),
            pl.BlockSpec((ka, kb), lambda i: (0, 0)),
        ],
        out_specs=pl.BlockSpec((_BLK, kb), lambda i: (i, 0)),
        out_shape=jax.ShapeDtypeStruct((n, kb), jnp.float32),
    )(a, b)


def _stage2_body(p_ref, add_ref, m1_ref, m2_ref, out_ref):
    h = jnp.dot(p_ref[...], m1_ref[...],
                preferred_element_type=jnp.float32) + add_ref[...]
    out_ref[...] = jnp.dot(h, m2_ref[...], preferred_element_type=jnp.float32)


def _stage2(pooled, add, m1, m2):
    n, hd = pooled.shape
    d = m1.shape[1]
    kb = m2.shape[1]
    return pl.pallas_call(
        _stage2_body,
        grid=(n // _BLK,),
        in_specs=[
            pl.BlockSpec((_BLK, hd), lambda i: (i, 0)),
            pl.BlockSpec((_BLK, d), lambda i: (i, 0)),
            pl.BlockSpec((hd, d), lambda i: (0, 0)),
            pl.BlockSpec((d, kb), lambda i: (0, 0)),
        ],
        out_specs=pl.BlockSpec((_BLK, kb), lambda i: (i, 0)),
        out_shape=jax.ShapeDtypeStruct((n, kb), jnp.float32),
    )(pooled, add, m1, m2)


def _final_body(p_ref, add_ref, m1_ref, out_ref):
    out_ref[...] = jnp.dot(p_ref[...], m1_ref[...],
                           preferred_element_type=jnp.float32) + add_ref[...]


def _final(pooled, add, m1):
    n, hd = pooled.shape
    d = m1.shape[1]
    return pl.pallas_call(
        _final_body,
        grid=(n // _BLK,),
        in_specs=[
            pl.BlockSpec((_BLK, hd), lambda i: (i, 0)),
            pl.BlockSpec((_BLK, d), lambda i: (i, 0)),
            pl.BlockSpec((hd, d), lambda i: (0, 0)),
        ],
        out_specs=pl.BlockSpec((_BLK, d), lambda i: (i, 0)),
        out_shape=jax.ShapeDtypeStruct((n, d), jnp.float32),
    )(pooled, add, m1)


def _sc_pool(yn, ys, gflat):
    n, h = ys.shape
    info = plsc.get_sparse_core_info()
    nc, ns = info.num_cores, info.num_subcores
    nw = nc * ns
    nodes_per_w = n // nw
    c_nodes = 8
    n_chunks = nodes_per_w // c_nodes
    mesh = plsc.VectorSubcoreMesh(core_axis_name="c", subcore_axis_name="s")

    @functools.partial(
        pl.kernel,
        out_type=jax.ShapeDtypeStruct((n, h), jnp.float32),
        mesh=mesh,
        scratch_types=[
            pltpu.VMEM((c_nodes * _K,), jnp.int32),
            pltpu.VMEM((c_nodes * _K,), jnp.int32),
            pltpu.VMEM((c_nodes * _K, h), jnp.float32),
            pltpu.VMEM((c_nodes * _K, h), jnp.float32),
            pltpu.VMEM((c_nodes, h), jnp.float32),
            pltpu.VMEM((c_nodes, h), jnp.float32),
            pltpu.SemaphoreType.DMA,
            pltpu.SemaphoreType.DMA,
        ],
    )
    def k(yn_hbm, ys_hbm, g_hbm, out_hbm,
          idx0, idx1, rows0, rows1, ys_v, out_v, sem0, sem1):
        wid = lax.axis_index("s") * nc + lax.axis_index("c")
        base_node = wid * nodes_per_w
        idx = (idx0, idx1)
        rows = (rows0, rows1)
        sems = (sem0, sem1)

        def issue(ci, b):
            node0 = base_node + ci * c_nodes
            pltpu.sync_copy(g_hbm.at[pl.ds(node0 * _K, c_nodes * _K)], idx[b])
            pltpu.async_copy(yn_hbm.at[idx[b]], rows[b], sems[b])

        issue(0, 0)

        def pair_body(g, carry):
            for b in range(2):
                ci = 2 * g + b

                @pl.when(ci + 1 < n_chunks)
                def _():
                    issue(ci + 1, 1 - b)

                pltpu.make_async_copy(yn_hbm.at[idx[b]], rows[b],
                                      sems[b]).wait()
                node0 = base_node + ci * c_nodes
                pltpu.sync_copy(ys_hbm.at[pl.ds(node0, c_nodes)], ys_v)

                def node_body(ni, inner, rows_v=rows[b]):
                    for c in range(h // 16):
                        sl = pl.ds(c * 16, 16)
                        yv = ys_v[ni, sl]
                        acc = jnp.zeros((16,), jnp.float32)
                        for j in range(_K):
                            v = rows_v[ni * _K + j, sl]
                            acc = acc + jnp.clip(v + yv, -1.0, 1.0)
                        out_v[ni, sl] = acc * (1.0 / _K)
                    return inner

                lax.fori_loop(0, c_nodes, node_body, 0)
                pltpu.sync_copy(out_v, out_hbm.at[pl.ds(node0, c_nodes)])
            return carry

        lax.fori_loop(0, n_chunks // 2, pair_body, 0)

    return k(yn, ys, gflat)


def kernel(x, w_0, w2_0, w_1, w2_1):
    n, d = x.shape
    hid = w_0.shape[0]

    w0cat = jnp.concatenate(
        [w_0[:, :d].T, w_0[:, d:].T, w2_0[:, hid:].T], axis=1)
    out0 = w2_0.shape[0]
    w1cat = jnp.concatenate(
        [w_1[:, :out0].T, w_1[:, out0:].T, w2_1[:, hid:].T], axis=1)
    w2p0t = w2_0[:, :hid].T
    w2p1t = w2_1[:, :hid].T

    graph = _knn(x, x.T)
    gflat = graph.reshape(-1)

    y0 = _mm(x, w0cat)
    pooled0 = _sc_pool(y0[:, :hid], y0[:, hid:2 * hid], gflat)
    y1 = _stage2(pooled0, y0[:, 2 * hid:], w2p0t, w1cat)
    pooled1 = _sc_pool(y1[:, :hid], y1[:, hid:2 * hid], gflat)
    return _final(pooled1, y1[:, 2 * hid:], w2p1t)


# speedup vs baseline: 6.0229x; 1.0045x over previous
import functools

import jax
import jax.numpy as jnp
from jax import lax
from jax.experimental import pallas as pl
from jax.experimental.pallas import tpu as pltpu
from jax.experimental.pallas import tpu_sc as plsc

_N = 4096
_K = 16
_BLK = 256


def _knn_body(x_ref, xt_ref, out_ref):
    xb = x_ref[...]
    xt = xt_ref[...]
    g = jnp.dot(xb, xt, preferred_element_type=jnp.float32)
    sqi = jnp.sum(xb * xb, axis=1, keepdims=True)
    sqj = jnp.sum(xt * xt, axis=0, keepdims=True)
    d2 = jnp.maximum(sqi + sqj - 2.0 * g, 0.0)
    colid = lax.broadcasted_iota(jnp.int32, d2.shape, 1)
    cols = []
    for _ in range(_K):
        m = jnp.min(d2, axis=1, keepdims=True)
        idx = jnp.min(jnp.where(d2 == m, colid, _N), axis=1, keepdims=True)
        cols.append(idx)
        d2 = jnp.where(colid == idx, jnp.float32(jnp.inf), d2)
    out_ref[...] = jnp.concatenate(cols, axis=1)


def _knn(x, xt):
    n, d = x.shape
    return pl.pallas_call(
        _knn_body,
        grid=(n // _BLK,),
        in_specs=[
            pl.BlockSpec((_BLK, d), lambda i: (i, 0)),
            pl.BlockSpec((d, n), lambda i: (0, 0)),
        ],
        out_specs=pl.BlockSpec((_BLK, _K), lambda i: (i, 0)),
        out_shape=jax.ShapeDtypeStruct((n, _K), jnp.int32),
    )(x, xt)


def _mm_body(a_ref, b_ref, out_ref):
    out_ref[...] = jnp.dot(a_ref[...], b_ref[...],
                           preferred_element_type=jnp.float32)


def _mm(a, b):
    n, ka = a.shape
    kb = b.shape[1]
    return pl.pallas_call(
        _mm_body,
        grid=(n // _BLK,),
        in_specs=[
            pl.BlockSpec((_BLK, ka), lambda i: (i, 0)),
            pl.BlockSpec((ka, kb), lambda i: (0, 0)),
        ],
        out_specs=pl.BlockSpec((_BLK, kb), lambda i: (i, 0)),
        out_shape=jax.ShapeDtypeStruct((n, kb), jnp.float32),
    )(a, b)


def _stage2_body(p_ref, add_ref, m1_ref, m2_ref, out_ref):
    h = jnp.dot(p_ref[...], m1_ref[...],
                preferred_element_type=jnp.float32) + add_ref[...]
    out_ref[...] = jnp.dot(h, m2_ref[...], preferred_element_type=jnp.float32)


def _stage2(pooled, add, m1, m2):
    n, hd = pooled.shape
    d = m1.shape[1]
    kb = m2.shape[1]
    return pl.pallas_call(
        _stage2_body,
        grid=(n // _BLK,),
        in_specs=[
            pl.BlockSpec((_BLK, hd), lambda i: (i, 0)),
            pl.BlockSpec((_BLK, d), lambda i: (i, 0)),
            pl.BlockSpec((hd, d), lambda i: (0, 0)),
            pl.BlockSpec((d, kb), lambda i: (0, 0)),
        ],
        out_specs=pl.BlockSpec((_BLK, kb), lambda i: (i, 0)),
        out_shape=jax.ShapeDtypeStruct((n, kb), jnp.float32),
    )(pooled, add, m1, m2)


def _final_body(p_ref, add_ref, m1_ref, out_ref):
    out_ref[...] = jnp.dot(p_ref[...], m1_ref[...],
                           preferred_element_type=jnp.float32) + add_ref[...]


def _final(pooled, add, m1):
    n, hd = pooled.shape
    d = m1.shape[1]
    return pl.pallas_call(
        _final_body,
        grid=(n // _BLK,),
        in_specs=[
            pl.BlockSpec((_BLK, hd), lambda i: (i, 0)),
            pl.BlockSpec((_BLK, d), lambda i: (i, 0)),
            pl.BlockSpec((hd, d), lambda i: (0, 0)),
        ],
        out_specs=pl.BlockSpec((_BLK, d), lambda i: (i, 0)),
        out_shape=jax.ShapeDtypeStruct((n, d), jnp.float32),
    )(pooled, add, m1)


def _sc_pool(yn, ys, gflat):
    n, h = ys.shape
    info = plsc.get_sparse_core_info()
    nc, ns = info.num_cores, info.num_subcores
    nw = nc * ns
    nodes_per_w = n // nw
    c_nodes = 8
    n_chunks = nodes_per_w // c_nodes
    mesh = plsc.VectorSubcoreMesh(core_axis_name="c", subcore_axis_name="s")

    @functools.partial(
        pl.kernel,
        out_type=jax.ShapeDtypeStruct((n, h), jnp.float32),
        mesh=mesh,
        scratch_types=[
            pltpu.VMEM((c_nodes * _K,), jnp.int32),
            pltpu.VMEM((c_nodes * _K,), jnp.int32),
            pltpu.VMEM((c_nodes * _K, h), jnp.float32),
            pltpu.VMEM((c_nodes * _K, h), jnp.float32),
            pltpu.VMEM((c_nodes, h), jnp.float32),
            pltpu.VMEM((c_nodes, h), jnp.float32),
            pltpu.VMEM((c_nodes, h), jnp.float32),
            pltpu.SemaphoreType.DMA,
            pltpu.SemaphoreType.DMA,
            pltpu.SemaphoreType.DMA,
            pltpu.SemaphoreType.DMA,
        ],
    )
    def k(yn_hbm, ys_hbm, g_hbm, out_hbm,
          idx0, idx1, rows0, rows1, ys_v, out0, out1,
          sem0, sem1, osem0, osem1):
        wid = lax.axis_index("s") * nc + lax.axis_index("c")
        base_node = wid * nodes_per_w
        idx = (idx0, idx1)
        rows = (rows0, rows1)
        sems = (sem0, sem1)
        outs = (out0, out1)
        osems = (osem0, osem1)

        def issue(ci, b):
            node0 = base_node + ci * c_nodes
            pltpu.sync_copy(g_hbm.at[pl.ds(node0 * _K, c_nodes * _K)], idx[b])
            pltpu.async_copy(yn_hbm.at[idx[b]], rows[b], sems[b])

        issue(0, 0)

        def pair_body(g, carry):
            for b in range(2):
                ci = 2 * g + b

                @pl.when(ci + 1 < n_chunks)
                def _():
                    issue(ci + 1, 1 - b)

                pltpu.make_async_copy(yn_hbm.at[idx[b]], rows[b],
                                      sems[b]).wait()
                node0 = base_node + ci * c_nodes
                pltpu.sync_copy(ys_hbm.at[pl.ds(node0, c_nodes)], ys_v)

                @pl.when(ci >= 2)
                def _():
                    prev0 = base_node + (ci - 2) * c_nodes
                    pltpu.make_async_copy(
                        outs[b], out_hbm.at[pl.ds(prev0, c_nodes)],
                        osems[b]).wait()

                def node_body(ni, inner, rows_v=rows[b], out_v=outs[b]):
                    for c in range(h // 16):
                        sl = pl.ds(c * 16, 16)
                        yv = ys_v[ni, sl]
                        acc = jnp.zeros((16,), jnp.float32)
                        for j in range(_K):
                            v = rows_v[ni * _K + j, sl]
                            acc = acc + jnp.clip(v + yv, -1.0, 1.0)
                        out_v[ni, sl] = acc * (1.0 / _K)
                    return inner

                lax.fori_loop(0, c_nodes, node_body, 0)
                pltpu.async_copy(outs[b], out_hbm.at[pl.ds(node0, c_nodes)],
                                 osems[b])
            return carry

        lax.fori_loop(0, n_chunks // 2, pair_body, 0)

        for b in range(2):
            last0 = base_node + (n_chunks - 2 + b) * c_nodes
            pltpu.make_async_copy(
                outs[b ^ (n_chunks & 1)],
                out_hbm.at[pl.ds(last0, c_nodes)], osems[b ^ (n_chunks & 1)],
            ).wait()

    return k(yn, ys, gflat)


def kernel(x, w_0, w2_0, w_1, w2_1):
    n, d = x.shape
    hid = w_0.shape[0]

    w0cat = jnp.concatenate(
        [w_0[:, :d].T, w_0[:, d:].T, w2_0[:, hid:].T], axis=1)
    out0 = w2_0.shape[0]
    w1cat = jnp.concatenate(
        [w_1[:, :out0].T, w_1[:, out0:].T, w2_1[:, hid:].T], axis=1)
    w2p0t = w2_0[:, :hid].T
    w2p1t = w2_1[:, :hid].T

    graph = _knn(x, x.T)
    gflat = graph.reshape(-1)

    y0 = _mm(x, w0cat)
    pooled0 = _sc_pool(y0[:, :hid], y0[:, hid:2 * hid], gflat)
    y1 = _stage2(pooled0, y0[:, 2 * hid:], w2p0t, w1cat)
    pooled1 = _sc_pool(y1[:, :hid], y1[:, hid:2 * hid], gflat)
    return _final(pooled1, y1[:, 2 * hid:], w2p1t)


# speedup vs baseline: 6.3534x; 1.0549x over previous
import functools

import jax
import jax.numpy as jnp
from jax import lax
from jax.experimental import pallas as pl
from jax.experimental.pallas import tpu as pltpu
from jax.experimental.pallas import tpu_sc as plsc

_N = 4096
_K = 16
_BLK = 256


def _knn_body(x_ref, xt_ref, out_ref):
    xb = x_ref[...]
    xt = xt_ref[...]
    g = jnp.dot(xb, xt, preferred_element_type=jnp.float32)
    sqi = jnp.sum(xb * xb, axis=1, keepdims=True)
    sqj = jnp.sum(xt * xt, axis=0, keepdims=True)
    d2 = jnp.maximum(sqi + sqj - 2.0 * g, 0.0)
    colid = lax.broadcasted_iota(jnp.int32, d2.shape, 1)
    cols = []
    for _ in range(_K):
        m = jnp.min(d2, axis=1, keepdims=True)
        idx = jnp.min(jnp.where(d2 == m, colid, _N), axis=1, keepdims=True)
        cols.append(idx)
        d2 = jnp.where(colid == idx, jnp.float32(jnp.inf), d2)
    out_ref[...] = jnp.concatenate(cols, axis=1)


def _knn(x, xt, row0, nrows):
    n, d = x.shape
    return pl.pallas_call(
        _knn_body,
        grid=(nrows // _BLK,),
        in_specs=[
            pl.BlockSpec((_BLK, d), lambda i: (i + row0 // _BLK, 0)),
            pl.BlockSpec((d, n), lambda i: (0, 0)),
        ],
        out_specs=pl.BlockSpec((_BLK, _K), lambda i: (i, 0)),
        out_shape=jax.ShapeDtypeStruct((nrows, _K), jnp.int32),
    )(x, xt)


def _mm_body(a_ref, b_ref, out_ref):
    out_ref[...] = jnp.dot(a_ref[...], b_ref[...],
                           preferred_element_type=jnp.float32)


def _mm(a, b):
    n, ka = a.shape
    kb = b.shape[1]
    return pl.pallas_call(
        _mm_body,
        grid=(n // _BLK,),
        in_specs=[
            pl.BlockSpec((_BLK, ka), lambda i: (i, 0)),
            pl.BlockSpec((ka, kb), lambda i: (0, 0)),
        ],
        out_specs=pl.BlockSpec((_BLK, kb), lambda i: (i, 0)),
        out_shape=jax.ShapeDtypeStruct((n, kb), jnp.float32),
    )(a, b)


def _stage2_body(p_ref, add_ref, m1_ref, m2_ref, out_ref):
    h = jnp.dot(p_ref[...], m1_ref[...],
                preferred_element_type=jnp.float32) + add_ref[...]
    out_ref[...] = jnp.dot(h, m2_ref[...], preferred_element_type=jnp.float32)


def _stage2(pooled, add, m1, m2):
    n, hd = pooled.shape
    d = m1.shape[1]
    kb = m2.shape[1]
    return pl.pallas_call(
        _stage2_body,
        grid=(n // _BLK,),
        in_specs=[
            pl.BlockSpec((_BLK, hd), lambda i: (i, 0)),
            pl.BlockSpec((_BLK, d), lambda i: (i, 0)),
            pl.BlockSpec((hd, d), lambda i: (0, 0)),
            pl.BlockSpec((d, kb), lambda i: (0, 0)),
        ],
        out_specs=pl.BlockSpec((_BLK, kb), lambda i: (i, 0)),
        out_shape=jax.ShapeDtypeStruct((n, kb), jnp.float32),
    )(pooled, add, m1, m2)


def _final_body(p_ref, add_ref, m1_ref, out_ref):
    out_ref[...] = jnp.dot(p_ref[...], m1_ref[...],
                           preferred_element_type=jnp.float32) + add_ref[...]


def _final(pooled, add, m1):
    n, hd = pooled.shape
    d = m1.shape[1]
    return pl.pallas_call(
        _final_body,
        grid=(n // _BLK,),
        in_specs=[
            pl.BlockSpec((_BLK, hd), lambda i: (i, 0)),
            pl.BlockSpec((_BLK, d), lambda i: (i, 0)),
            pl.BlockSpec((hd, d), lambda i: (0, 0)),
        ],
        out_specs=pl.BlockSpec((_BLK, d), lambda i: (i, 0)),
        out_shape=jax.ShapeDtypeStruct((n, d), jnp.float32),
    )(pooled, add, m1)


def _sc_pool(yn, ys, gflat):
    n, h = ys.shape
    info = plsc.get_sparse_core_info()
    nc, ns = info.num_cores, info.num_subcores
    nw = nc * ns
    nodes_per_w = n // nw
    c_nodes = 8
    n_chunks = nodes_per_w // c_nodes
    mesh = plsc.VectorSubcoreMesh(core_axis_name="c", subcore_axis_name="s")

    @functools.partial(
        pl.kernel,
        out_type=jax.ShapeDtypeStruct((n, h), jnp.float32),
        mesh=mesh,
        scratch_types=[
            pltpu.VMEM((c_nodes * _K,), jnp.int32),
            pltpu.VMEM((c_nodes * _K,), jnp.int32),
            pltpu.VMEM((c_nodes * _K, h), jnp.float32),
            pltpu.VMEM((c_nodes * _K, h), jnp.float32),
            pltpu.VMEM((c_nodes, h), jnp.float32),
            pltpu.VMEM((c_nodes, h), jnp.float32),
            pltpu.VMEM((c_nodes, h), jnp.float32),
            pltpu.SemaphoreType.DMA,
            pltpu.SemaphoreType.DMA,
            pltpu.SemaphoreType.DMA,
            pltpu.SemaphoreType.DMA,
        ],
    )
    def k(yn_hbm, ys_hbm, g_hbm, out_hbm,
          idx0, idx1, rows0, rows1, ys_v, out0, out1,
          sem0, sem1, osem0, osem1):
        wid = lax.axis_index("s") * nc + lax.axis_index("c")
        base_node = wid * nodes_per_w
        idx = (idx0, idx1)
        rows = (rows0, rows1)
        sems = (sem0, sem1)
        outs = (out0, out1)
        osems = (osem0, osem1)

        def issue(ci, b):
            node0 = base_node + ci * c_nodes
            pltpu.sync_copy(g_hbm.at[pl.ds(node0 * _K, c_nodes * _K)], idx[b])
            pltpu.async_copy(yn_hbm.at[idx[b]], rows[b], sems[b])

        issue(0, 0)

        def pair_body(g, carry):
            for b in range(2):
                ci = 2 * g + b

                @pl.when(ci + 1 < n_chunks)
                def _():
                    issue(ci + 1, 1 - b)

                pltpu.make_async_copy(yn_hbm.at[idx[b]], rows[b],
                                      sems[b]).wait()
                node0 = base_node + ci * c_nodes
                pltpu.sync_copy(ys_hbm.at[pl.ds(node0, c_nodes)], ys_v)

                @pl.when(ci >= 2)
                def _():
                    prev0 = base_node + (ci - 2) * c_nodes
                    pltpu.make_async_copy(
                        outs[b], out_hbm.at[pl.ds(prev0, c_nodes)],
                        osems[b]).wait()

                def node_body(ni, inner, rows_v=rows[b], out_v=outs[b]):
                    for c in range(h // 16):
                        sl = pl.ds(c * 16, 16)
                        yv = ys_v[ni, sl]
                        acc = jnp.zeros((16,), jnp.float32)
                        for j in range(_K):
                            v = rows_v[ni * _K + j, sl]
                            acc = acc + jnp.clip(v + yv, -1.0, 1.0)
                        out_v[ni, sl] = acc * (1.0 / _K)
                    return inner

                lax.fori_loop(0, c_nodes, node_body, 0)
                pltpu.async_copy(outs[b], out_hbm.at[pl.ds(node0, c_nodes)],
                                 osems[b])
            return carry

        lax.fori_loop(0, n_chunks // 2, pair_body, 0)

        for b in range(2):
            last0 = base_node + (n_chunks - 2 + b) * c_nodes
            pltpu.make_async_copy(
                outs[b ^ (n_chunks & 1)],
                out_hbm.at[pl.ds(last0, c_nodes)], osems[b ^ (n_chunks & 1)],
            ).wait()

    return k(yn, ys, gflat)


def kernel(x, w_0, w2_0, w_1, w2_1):
    n, d = x.shape
    hid = w_0.shape[0]

    w0cat = jnp.concatenate(
        [w_0[:, :d].T, w_0[:, d:].T, w2_0[:, hid:].T], axis=1)
    out0 = w2_0.shape[0]
    w1cat = jnp.concatenate(
        [w_1[:, :out0].T, w_1[:, out0:].T, w2_1[:, hid:].T], axis=1)
    w2p0t = w2_0[:, :hid].T
    w2p1t = w2_1[:, :hid].T

    xt = x.T
    half = n // 2
    y0 = _mm(x, w0cat)
    yn0 = y0[:, :hid]
    graph_a = _knn(x, xt, 0, half)
    pooled_a = _sc_pool(yn0, y0[:half, hid:2 * hid], graph_a.reshape(-1))
    graph_b = _knn(x, xt, half, half)
    pooled_b = _sc_pool(yn0, y0[half:, hid:2 * hid], graph_b.reshape(-1))
    pooled0 = jnp.concatenate([pooled_a, pooled_b], axis=0)
    gflat = jnp.concatenate([graph_a, graph_b], axis=0).reshape(-1)

    y1 = _stage2(pooled0, y0[:, 2 * hid:], w2p0t, w1cat)
    pooled1 = _sc_pool(y1[:, :hid], y1[:, hid:2 * hid], gflat)
    return _final(pooled1, y1[:, 2 * hid:], w2p1t)


# speedup vs baseline: 6.4411x; 1.0138x over previous
import functools

import jax
import jax.numpy as jnp
from jax import lax
from jax.experimental import pallas as pl
from jax.experimental.pallas import tpu as pltpu
from jax.experimental.pallas import tpu_sc as plsc

_N = 4096
_K = 16
_BLK = 256


def _knn_body(x_ref, xt_ref, out_ref):
    xb = x_ref[...]
    xt = xt_ref[...]
    g = jnp.dot(xb, xt, preferred_element_type=jnp.float32)
    sqi = jnp.sum(xb * xb, axis=1, keepdims=True)
    sqj = jnp.sum(xt * xt, axis=0, keepdims=True)
    d2 = jnp.maximum(sqi + sqj - 2.0 * g, 0.0)
    colid = lax.broadcasted_iota(jnp.int32, d2.shape, 1)
    cols = []
    for _ in range(_K):
        m = jnp.min(d2, axis=1, keepdims=True)
        idx = jnp.min(jnp.where(d2 == m, colid, _N), axis=1, keepdims=True)
        cols.append(idx)
        d2 = jnp.where(colid == idx, jnp.float32(jnp.inf), d2)
    out_ref[...] = jnp.concatenate(cols, axis=1)


def _knn(x, xt, row0, nrows):
    n, d = x.shape
    return pl.pallas_call(
        _knn_body,
        grid=(nrows // _BLK,),
        in_specs=[
            pl.BlockSpec((_BLK, d), lambda i: (i + row0 // _BLK, 0)),
            pl.BlockSpec((d, n), lambda i: (0, 0)),
        ],
        out_specs=pl.BlockSpec((_BLK, _K), lambda i: (i, 0)),
        out_shape=jax.ShapeDtypeStruct((nrows, _K), jnp.int32),
    )(x, xt)


def _mm_body(a_ref, b_ref, out_ref):
    out_ref[...] = jnp.dot(a_ref[...], b_ref[...],
                           preferred_element_type=jnp.float32)


def _mm(a, b):
    n, ka = a.shape
    kb = b.shape[1]
    return pl.pallas_call(
        _mm_body,
        grid=(n // _BLK,),
        in_specs=[
            pl.BlockSpec((_BLK, ka), lambda i: (i, 0)),
            pl.BlockSpec((ka, kb), lambda i: (0, 0)),
        ],
        out_specs=pl.BlockSpec((_BLK, kb), lambda i: (i, 0)),
        out_shape=jax.ShapeDtypeStruct((n, kb), jnp.float32),
    )(a, b)


def _stage2_body(p_ref, add_ref, m1_ref, m2_ref, out_ref):
    h = jnp.dot(p_ref[...], m1_ref[...],
                preferred_element_type=jnp.float32) + add_ref[...]
    out_ref[...] = jnp.dot(h, m2_ref[...], preferred_element_type=jnp.float32)


def _stage2(pooled, add, m1, m2):
    n, hd = pooled.shape
    d = m1.shape[1]
    kb = m2.shape[1]
    return pl.pallas_call(
        _stage2_body,
        grid=(n // _BLK,),
        in_specs=[
            pl.BlockSpec((_BLK, hd), lambda i: (i, 0)),
            pl.BlockSpec((_BLK, d), lambda i: (i, 0)),
            pl.BlockSpec((hd, d), lambda i: (0, 0)),
            pl.BlockSpec((d, kb), lambda i: (0, 0)),
        ],
        out_specs=pl.BlockSpec((_BLK, kb), lambda i: (i, 0)),
        out_shape=jax.ShapeDtypeStruct((n, kb), jnp.float32),
    )(pooled, add, m1, m2)


def _final_body(p_ref, add_ref, m1_ref, out_ref):
    out_ref[...] = jnp.dot(p_ref[...], m1_ref[...],
                           preferred_element_type=jnp.float32) + add_ref[...]


def _final(pooled, add, m1):
    n, hd = pooled.shape
    d = m1.shape[1]
    return pl.pallas_call(
        _final_body,
        grid=(n // _BLK,),
        in_specs=[
            pl.BlockSpec((_BLK, hd), lambda i: (i, 0)),
            pl.BlockSpec((_BLK, d), lambda i: (i, 0)),
            pl.BlockSpec((hd, d), lambda i: (0, 0)),
        ],
        out_specs=pl.BlockSpec((_BLK, d), lambda i: (i, 0)),
        out_shape=jax.ShapeDtypeStruct((n, d), jnp.float32),
    )(pooled, add, m1)


def _sc_pool(yn, ys, gflat):
    n, h = ys.shape
    info = plsc.get_sparse_core_info()
    nc, ns = info.num_cores, info.num_subcores
    nw = nc * ns
    nodes_per_w = n // nw
    c_nodes = 8
    n_chunks = nodes_per_w // c_nodes
    mesh = plsc.VectorSubcoreMesh(core_axis_name="c", subcore_axis_name="s")

    @functools.partial(
        pl.kernel,
        out_type=jax.ShapeDtypeStruct((n, h), jnp.float32),
        mesh=mesh,
        scratch_types=[
            pltpu.VMEM((c_nodes * _K,), jnp.int32),
            pltpu.VMEM((c_nodes * _K,), jnp.int32),
            pltpu.VMEM((c_nodes * _K, h), jnp.float32),
            pltpu.VMEM((c_nodes * _K, h), jnp.float32),
            pltpu.VMEM((c_nodes, h), jnp.float32),
            pltpu.VMEM((c_nodes, h), jnp.float32),
            pltpu.VMEM((c_nodes, h), jnp.float32),
            pltpu.SemaphoreType.DMA,
            pltpu.SemaphoreType.DMA,
            pltpu.SemaphoreType.DMA,
            pltpu.SemaphoreType.DMA,
        ],
    )
    def k(yn_hbm, ys_hbm, g_hbm, out_hbm,
          idx0, idx1, rows0, rows1, ys_v, out0, out1,
          sem0, sem1, osem0, osem1):
        wid = lax.axis_index("s") * nc + lax.axis_index("c")
        base_node = wid * nodes_per_w
        idx = (idx0, idx1)
        rows = (rows0, rows1)
        sems = (sem0, sem1)
        outs = (out0, out1)
        osems = (osem0, osem1)

        def issue(ci, b):
            node0 = base_node + ci * c_nodes
            pltpu.sync_copy(g_hbm.at[pl.ds(node0 * _K, c_nodes * _K)], idx[b])
            pltpu.async_copy(yn_hbm.at[idx[b]], rows[b], sems[b])

        issue(0, 0)

        def pair_body(g, carry):
            for b in range(2):
                ci = 2 * g + b

                @pl.when(ci + 1 < n_chunks)
                def _():
                    issue(ci + 1, 1 - b)

                pltpu.make_async_copy(yn_hbm.at[idx[b]], rows[b],
                                      sems[b]).wait()
                node0 = base_node + ci * c_nodes
                pltpu.sync_copy(ys_hbm.at[pl.ds(node0, c_nodes)], ys_v)

                @pl.when(ci >= 2)
                def _():
                    prev0 = base_node + (ci - 2) * c_nodes
                    pltpu.make_async_copy(
                        outs[b], out_hbm.at[pl.ds(prev0, c_nodes)],
                        osems[b]).wait()

                def node_body(ni, inner, rows_v=rows[b], out_v=outs[b]):
                    for c in range(h // 16):
                        sl = pl.ds(c * 16, 16)
                        yv = ys_v[ni, sl]
                        acc = jnp.zeros((16,), jnp.float32)
                        for j in range(_K):
                            v = rows_v[ni * _K + j, sl]
                            acc = acc + jnp.clip(v + yv, -1.0, 1.0)
                        out_v[ni, sl] = acc * (1.0 / _K)
                    return inner

                lax.fori_loop(0, c_nodes, node_body, 0)
                pltpu.async_copy(outs[b], out_hbm.at[pl.ds(node0, c_nodes)],
                                 osems[b])
            return carry

        lax.fori_loop(0, n_chunks // 2, pair_body, 0)

        for b in range(2):
            last0 = base_node + (n_chunks - 2 + b) * c_nodes
            pltpu.make_async_copy(
                outs[b ^ (n_chunks & 1)],
                out_hbm.at[pl.ds(last0, c_nodes)], osems[b ^ (n_chunks & 1)],
            ).wait()

    return k(yn, ys, gflat)


def kernel(x, w_0, w2_0, w_1, w2_1):
    n, d = x.shape
    hid = w_0.shape[0]

    w0cat = jnp.concatenate(
        [w_0[:, :d].T, w_0[:, d:].T, w2_0[:, hid:].T], axis=1)
    out0 = w2_0.shape[0]
    w1cat = jnp.concatenate(
        [w_1[:, :out0].T, w_1[:, out0:].T, w2_1[:, hid:].T], axis=1)
    w2p0t = w2_0[:, :hid].T
    w2p1t = w2_1[:, :hid].T

    xt = x.T
    q = n // 4
    y0 = _mm(x, w0cat)
    yn0 = y0[:, :hid]
    graphs = []
    pooled_parts = []
    for p in range(4):
        g = _knn(x, xt, p * q, q)
        graphs.append(g)
        pooled_parts.append(
            _sc_pool(yn0, y0[p * q:(p + 1) * q, hid:2 * hid], g.reshape(-1)))
    pooled0 = jnp.concatenate(pooled_parts, axis=0)
    gflat = jnp.concatenate(graphs, axis=0).reshape(-1)

    y1 = _stage2(pooled0, y0[:, 2 * hid:], w2p0t, w1cat)
    pooled1 = _sc_pool(y1[:, :hid], y1[:, hid:2 * hid], gflat)
    return _final(pooled1, y1[:, 2 * hid:], w2p1t)
